# Initial kernel scaffold; baseline (speedup 1.0000x reference)
import jax
import jax.numpy as jnp
from jax.experimental import pallas as pl


def kernel(pred0, pred1, pred2, targets):
    raise NotImplementedError("write your pallas kernel here")


# speedup vs baseline: 1.3469x; 1.3469x over previous
import functools

import numpy as np
import jax
import jax.numpy as jnp
from jax import lax
from jax.experimental import pallas as pl
from jax.experimental.pallas import tpu as pltpu
from jax.experimental.pallas import tpu_sc as plsc

_NC = 80
_IMG = 640
_NA = 3
_NB = 32
_NT = 1024
_N = 15360
_ANCH = np.array(
    [[10.0, 13.0], [16.0, 30.0], [33.0, 23.0], [30.0, 61.0], [62.0, 45.0],
     [59.0, 119.0], [116.0, 90.0], [156.0, 198.0], [373.0, 326.0]],
    dtype=np.float32)
_HWS = [(80, 80), (40, 40), (20, 20)]
_CELL_BASE = [0, _NB * 3 * 6400, _NB * 3 * 6400 + _NB * 3 * 1600]
_DTOT = _NB * 3 * (6400 + 1600 + 400)
_SENT = 4.0e6

_NW = 32
_DCH = _DTOT // _NW
_GTOT = 88 * _N
_GPW = _GTOT // _NW
_GCH = _GPW // 4


def _prep_body(tt_ref, idx_ref, meta_ref, *, H, W, anchors, cell_base):
    HW = H * W
    col = lax.broadcasted_iota(jnp.int32, (1, _N), 1)
    a = (col // _NT) % 3
    o = col // (3 * _NT)

    def tiled(r):
        row = tt_ref[r:r + 1, :]
        return jnp.concatenate([row] * 15, axis=1)

    bi = tiled(0)
    cls_f = tiled(1)
    gx = tiled(2) * W
    gy = tiled(3) * H
    gw = tiled(4) * W
    gh = tiled(5) * H

    af = a.astype(jnp.float32)
    aw = jnp.where(af == 0.0, anchors[0, 0],
                   jnp.where(af == 1.0, anchors[1, 0], anchors[2, 0]))
    ah = jnp.where(af == 0.0, anchors[0, 1],
                   jnp.where(af == 1.0, anchors[1, 1], anchors[2, 1]))
    rw = gw / aw
    rh = gh / ah
    fitf = jnp.where(
        jnp.maximum(jnp.maximum(rw, 1.0 / rw), jnp.maximum(rh, 1.0 / rh)) < 4.0,
        1.0, 0.0)
    gxi = W - gx
    gyi = H - gy

    def near(u):
        return jnp.where(u % 1.0 < 0.5, 1.0, 0.0) * jnp.where(u > 1.0, 1.0, 0.0)

    jk0, jk1, lm0, lm1 = near(gx), near(gy), near(gxi), near(gyi)
    jmf = jnp.where(o == 0, 1.0,
                    jnp.where(o == 1, jk0,
                              jnp.where(o == 2, jk1,
                                        jnp.where(o == 3, lm0, lm1))))
    validf = jmf * fitf
    valid = validf > 0.5
    ox = jnp.where(o == 1, 1.0, jnp.where(o == 3, -1.0, 0.0))
    oy = jnp.where(o == 2, 1.0, jnp.where(o == 4, -1.0, 0.0))
    gi0 = (gx - ox).astype(jnp.int32)
    gj0 = (gy - oy).astype(jnp.int32)
    gi = jnp.clip(gi0, 0, W - 1)
    gj = jnp.clip(gj0, 0, H - 1)
    bii = bi.astype(jnp.int32)
    clsi = cls_f.astype(jnp.int32)

    base = (bii * 255 + a * 85) * HW + gj * W + gi
    row = lax.broadcasted_iota(jnp.int32, (88, _N), 0)
    chan = jnp.where(row == 85, 5 + jnp.broadcast_to(clsi, (88, _N)),
                     jnp.where(row > 85, 0, row))
    idx_ref[...] = jnp.broadcast_to(base, (88, _N)) + chan * HW

    cellf = jnp.where(
        valid,
        (((bii * 3 + a) * H + gj) * W + gi + cell_base).astype(jnp.float32),
        _SENT)
    meta_ref[...] = jnp.concatenate(
        [gx - gi0.astype(jnp.float32), gy - gj0.astype(jnp.float32),
         gw, gh, validf, cellf,
         jnp.broadcast_to(aw, (1, _N)), jnp.broadcast_to(ah, (1, _N))],
        axis=0)


def _prep(tt_pad, s):
    H, W = _HWS[s]
    stride = _IMG // W
    anchors = _ANCH[s * 3:(s + 1) * 3] / stride
    return pl.pallas_call(
        functools.partial(_prep_body, H=H, W=W, anchors=anchors,
                          cell_base=_CELL_BASE[s]),
        out_shape=[
            jax.ShapeDtypeStruct((88, _N), jnp.int32),
            jax.ShapeDtypeStruct((8, _N), jnp.float32),
        ],
        in_specs=[pl.BlockSpec((8, _NT), lambda: (0, 0))],
        out_specs=[pl.BlockSpec((88, _N), lambda: (0, 0)),
                   pl.BlockSpec((8, _N), lambda: (0, 0))],
    )(tt_pad)


def _obj_body(p_ref, out_ref):
    i = pl.program_id(0)

    @pl.when(i == 0)
    def _():
        out_ref[...] = jnp.zeros_like(out_ref)

    x = p_ref[0, 0, :, :]
    s = jnp.sum(jnp.logaddexp(0.0, x))
    r = lax.broadcasted_iota(jnp.int32, (8, 128), 0)
    c = lax.broadcasted_iota(jnp.int32, (8, 128), 1)
    out_ref[...] += jnp.where((r == 0) & (c == 0), s, 0.0)


def _obj_sum(p, s):
    H, W = _HWS[s]
    return pl.pallas_call(
        _obj_body,
        grid=(_NB * 3,),
        out_shape=jax.ShapeDtypeStruct((8, 128), jnp.float32),
        in_specs=[pl.BlockSpec((1, 1, H, W),
                               lambda i: (i // 3, (i % 3) * 85 + 4, 0, 0))],
        out_specs=pl.BlockSpec((8, 128), lambda i: (0, 0)),
    )(p)


def _sc_body(p0, p1, p2, i0, i1, i2, m0, m1, m2,
             o0, o1, o2, win, idxbuf, databuf, cell0, cell1, cell2,
             dense, winbuf, gsem):
    wid = lax.axis_index("s") * 2 + lax.axis_index("c")
    preds = (p0, p1, p2)
    idxs = (i0, i1, i2)
    outs = (o0, o1, o2)
    metas = (m0, m1, m2)
    lane = lax.iota(jnp.int32, 16)

    w0 = wid * _GPW
    for s_ in range(3):
        def gchunk(k, _, s_=s_):
            off = w0 + k * _GCH
            pltpu.sync_copy(idxs[s_].at[pl.ds(off, _GCH)], idxbuf)
            pltpu.async_copy(preds[s_].at[idxbuf], databuf, gsem).wait()
            pltpu.sync_copy(databuf, outs[s_].at[pl.ds(off, _GCH)])
            return _

        lax.fori_loop(0, _GPW // _GCH, gchunk, 0, unroll=True)

    cells = (cell0, cell1, cell2)
    for s_ in range(3):
        pltpu.sync_copy(metas[s_].at[5, :], cells[s_])

    def ms(i, _):
        dense[pl.ds(i * 16, 16)] = jnp.full((16,), -1, jnp.int32)
        return _

    lax.fori_loop(0, _DCH // 16, ms, 0)

    wbase = wid * _DCH
    for s_ in range(3):
        def p1b(i, _, s_=s_):
            c = cells[s_][pl.ds(i * 16, 16)].astype(jnp.int32) - wbase
            m = (c >= 0) & (c < _DCH)
            cs = jnp.where(m, c, 0)
            plsc.store_scatter(dense, [cs], i * 16 + lane, mask=m)
            return _

        lax.fori_loop(0, _N // 16, p1b, 0)

    for s_ in range(3):
        def p2b(i, _, s_=s_):
            c = cells[s_][pl.ds(i * 16, 16)].astype(jnp.int32) - wbase
            m = (c >= 0) & (c < _DCH)
            cs = jnp.where(m, c, 0)
            w = plsc.load_gather(dense, [cs], mask=m)
            isw = m & (w == i * 16 + lane)
            winbuf[pl.ds(i * 16, 16)] = jnp.where(isw, 1.0, 0.0)
            return _

        lax.fori_loop(0, _N // 16, p2b, 0)
        pltpu.sync_copy(winbuf, win.at[wid, pl.ds(s_ * _N, _N)])


def _sc_call(p0f, p1f, p2f, idx0, idx1, idx2, meta0, meta1, meta2):
    mesh = plsc.VectorSubcoreMesh(core_axis_name="c", subcore_axis_name="s",
                                  num_cores=2, num_subcores=16)
    f = pl.kernel(
        _sc_body,
        out_type=[
            jax.ShapeDtypeStruct((_GTOT,), jnp.float32),
            jax.ShapeDtypeStruct((_GTOT,), jnp.float32),
            jax.ShapeDtypeStruct((_GTOT,), jnp.float32),
            jax.ShapeDtypeStruct((_NW, 3 * _N), jnp.float32),
        ],
        mesh=mesh,
        scratch_types=[
            pltpu.VMEM((_GCH,), jnp.int32),
            pltpu.VMEM((_GCH,), jnp.float32),
            pltpu.VMEM((_N,), jnp.float32),
            pltpu.VMEM((_N,), jnp.float32),
            pltpu.VMEM((_N,), jnp.float32),
            pltpu.VMEM((_DCH,), jnp.int32),
            pltpu.VMEM((_N,), jnp.float32),
            pltpu.SemaphoreType.DMA,
        ],
        compiler_params=pltpu.CompilerParams(needs_layout_passes=False),
    )
    return f(p0f, p1f, p2f, idx0, idx1, idx2, meta0, meta1, meta2)


def _atan_pos(x):
    inv = x > 1.0
    y = jnp.where(inv, 1.0 / x, x)
    z = y * y
    p = y * (0.9998660 + z * (-0.3302995 + z * (0.1801410 + z *
             (-0.0851330 + z * 0.0208351))))
    return jnp.where(inv, (np.pi / 2) - p, p)


def _math_body(g0, g1, g2, m0, m1, m2, w0, w1, w2, out_ref):
    i = pl.program_id(0)

    @pl.when(i == 0)
    def _():
        out_ref[...] = jnp.zeros_like(out_ref)

    acc = jnp.zeros((8, 128), jnp.float32)
    r_i = lax.broadcasted_iota(jnp.int32, (8, 128), 0)
    c_i = lax.broadcasted_iota(jnp.int32, (8, 128), 1)
    for s_, (g, mt, w) in enumerate(((g0, m0, w0), (g1, m1, w1),
                                     (g2, m2, w2))):
        tbx = mt[0:1, :]
        tby = mt[1:2, :]
        tbw = mt[2:3, :]
        tbh = mt[3:4, :]
        valid = mt[4:5, :]
        aw = mt[6:7, :]
        ah = mt[7:8, :]

        b1x = jax.nn.sigmoid(g[0:1, :])
        b1y = jax.nn.sigmoid(g[1:2, :])
        w1_ = jnp.exp(g[2:3, :]) * aw
        h1_ = jnp.exp(g[3:4, :]) * ah
        ps4 = g[4:5, :]
        pstc = g[85:86, :]

        b1x1 = b1x - w1_ / 2
        b1x2 = b1x + w1_ / 2
        b1y1 = b1y - h1_ / 2
        b1y2 = b1y + h1_ / 2
        b2x1 = tbx - tbw / 2
        b2x2 = tbx + tbw / 2
        b2y1 = tby - tbh / 2
        b2y2 = tby + tbh / 2
        inter = (jnp.maximum(jnp.minimum(b1x2, b2x2) -
                             jnp.maximum(b1x1, b2x1), 0.0) *
                 jnp.maximum(jnp.minimum(b1y2, b2y2) -
                             jnp.maximum(b1y1, b2y1), 0.0))
        union = w1_ * h1_ + tbw * tbh - inter + 1e-16
        iou0 = inter / union
        cw = jnp.maximum(b1x2, b2x2) - jnp.minimum(b1x1, b2x1)
        ch = jnp.maximum(b1y2, b2y2) - jnp.minimum(b1y1, b2y1)
        c2 = cw * cw + ch * ch + 1e-16
        rho2 = ((b2x1 + b2x2 - b1x1 - b1x2) ** 2 +
                (b2y1 + b2y2 - b1y1 - b1y2) ** 2) / 4
        v = (4.0 / 3.14159 ** 2) * (_atan_pos(tbw / tbh) -
                                    _atan_pos(w1_ / h1_)) ** 2
        alpha = v / (v - iou0 + (1.0 + 1e-16))
        iou = iou0 - (rho2 / c2 + v * alpha)

        box_p = jnp.sum((1.0 - iou) * valid)
        cnt_p = jnp.sum(valid)
        cls_row = jnp.sum(jnp.logaddexp(0.0, g[5:85, :]), axis=0,
                          keepdims=True) - pstc
        cls_p = jnp.sum(cls_row * valid)
        wsum = jnp.sum(w[...], axis=0, keepdims=True)
        win_p = jnp.sum(wsum * jnp.maximum(iou, 0.0) * ps4)

        vals = jnp.where(c_i == 0, box_p,
                         jnp.where(c_i == 1, cnt_p,
                                   jnp.where(c_i == 2, cls_p, win_p)))
        acc += jnp.where((r_i == s_) & (c_i < 4), vals, 0.0)

    out_ref[...] += acc


def _math(gs, ms, win):
    nblk = 15
    bw = _N // nblk
    return pl.pallas_call(
        _math_body,
        grid=(nblk,),
        out_shape=jax.ShapeDtypeStruct((8, 128), jnp.float32),
        in_specs=(
            [pl.BlockSpec((88, bw), lambda i: (0, i)) for _ in range(3)] +
            [pl.BlockSpec((8, bw), lambda i: (0, i)) for _ in range(3)] +
            [pl.BlockSpec((_NW, bw), lambda i, s_=s_: (0, s_ * nblk + i))
             for s_ in range(3)]),
        out_specs=pl.BlockSpec((8, 128), lambda i: (0, 0)),
    )(*gs, *ms, win, win, win)


def kernel(pred0, pred1, pred2, targets):
    preds = (pred0, pred1, pred2)
    tt = jnp.pad(targets.T, ((0, 2), (0, 0)))

    idxs, metas = [], []
    for s in range(3):
        idx, meta = _prep(tt, s)
        idxs.append(idx)
        metas.append(meta)

    objs = [_obj_sum(preds[s], s) for s in range(3)]

    g0, g1, g2, win = _sc_call(
        pred0.reshape(-1), pred1.reshape(-1), pred2.reshape(-1),
        *[x.reshape(-1) for x in idxs], *metas)

    gs = [g.reshape(88, _N) for g in (g0, g1, g2)]
    res = _math(gs, metas, win)

    lbox = jnp.float32(0.0)
    lobj = jnp.float32(0.0)
    lcls = jnp.float32(0.0)
    for s in range(3):
        H, W = _HWS[s]
        box_p, cnt, cls_p, win_p = res[s, 0], res[s, 1], res[s, 2], res[s, 3]
        lbox += box_p / cnt
        lcls += cls_p / (cnt * _NC)
        lobj += (objs[s][0, 0] - win_p) / (_NB * 3 * H * W)
    lbox *= 0.05
    lcls *= 0.5
    loss = lbox + lobj + lcls
    return loss, jnp.stack([lbox, lobj, lcls])


# speedup vs baseline: 2.2541x; 1.6736x over previous
import functools

import numpy as np
import jax
import jax.numpy as jnp
from jax import lax
from jax.experimental import pallas as pl
from jax.experimental.pallas import tpu as pltpu
from jax.experimental.pallas import tpu_sc as plsc

_NC = 80
_IMG = 640
_NB = 32
_NT = 1024
_N = 15360
_ANCH = np.array(
    [[10.0, 13.0], [16.0, 30.0], [33.0, 23.0], [30.0, 61.0], [62.0, 45.0],
     [59.0, 119.0], [116.0, 90.0], [156.0, 198.0], [373.0, 326.0]],
    dtype=np.float32)
_HWS = [(80, 80), (40, 40), (20, 20)]
_CELL_BASE = [0, _NB * 3 * 6400, _NB * 3 * 6400 + _NB * 3 * 1600]
_DTOT = _NB * 3 * (6400 + 1600 + 400)
_SENT = 4.0e6

_NW = 32
_DCH = _DTOT // _NW
_CK = 128
_PERMS = [(0, 2, 3, 1), (0, 2, 3, 1), (2, 3, 0, 1)]


def _repack_body(p_ref, tab_ref, obj_ref, *, bh, W):
    i = pl.program_id(0)
    j = pl.program_id(1)

    @pl.when((i == 0) & (j == 0))
    def _():
        obj_ref[...] = jnp.zeros_like(obj_ref)

    x = p_ref[0]
    x2 = x.reshape(bh * W, 255)
    tab_ref[...] = jnp.concatenate(
        [x2, jnp.zeros((bh * W, 1), jnp.float32)], axis=1)
    s = (jnp.sum(jnp.logaddexp(0.0, x2[:, 4:5])) +
         jnp.sum(jnp.logaddexp(0.0, x2[:, 89:90])) +
         jnp.sum(jnp.logaddexp(0.0, x2[:, 174:175])))
    r = lax.broadcasted_iota(jnp.int32, (8, 128), 0)
    c = lax.broadcasted_iota(jnp.int32, (8, 128), 1)
    obj_ref[...] += jnp.where((r == 0) & (c == 0), s, 0.0)


def _repack(p_cl, s):
    H, W = _HWS[s]
    d0, d1 = p_cl.shape[0], p_cl.shape[1]
    bh = 8 if d1 % 8 == 0 else 4
    R = d0 * d1 * p_cl.shape[2]
    return pl.pallas_call(
        functools.partial(_repack_body, bh=bh, W=p_cl.shape[2]),
        grid=(d0, d1 // bh),
        out_shape=[jax.ShapeDtypeStruct((R, 256), jnp.float32),
                   jax.ShapeDtypeStruct((8, 128), jnp.float32)],
        in_specs=[pl.BlockSpec((1, bh, p_cl.shape[2], 255),
                               lambda i, j: (i, j, 0, 0))],
        out_specs=[pl.BlockSpec((bh * p_cl.shape[2], 256),
                                lambda i, j, d1=d1, bh=bh:
                                (i * (d1 // bh) + j, 0)),
                   pl.BlockSpec((8, 128), lambda i, j: (0, 0))],
    )(p_cl)


def _prep_body(tt_ref, aux_ref, meta_ref, *, H, W, anchors, cell_base, border):
    col = lax.broadcasted_iota(jnp.int32, (1, _N), 1)
    a = (col // _NT) % 3
    o = col // (3 * _NT)

    def tiled(r):
        row = tt_ref[r:r + 1, :]
        return jnp.concatenate([row] * 15, axis=1)

    bi = tiled(0)
    cls_f = tiled(1)
    gx = tiled(2) * W
    gy = tiled(3) * H
    gw = tiled(4) * W
    gh = tiled(5) * H

    af = a.astype(jnp.float32)
    aw = jnp.where(af == 0.0, anchors[0, 0],
                   jnp.where(af == 1.0, anchors[1, 0], anchors[2, 0]))
    ah = jnp.where(af == 0.0, anchors[0, 1],
                   jnp.where(af == 1.0, anchors[1, 1], anchors[2, 1]))
    rw = gw / aw
    rh = gh / ah
    fitf = jnp.where(
        jnp.maximum(jnp.maximum(rw, 1.0 / rw), jnp.maximum(rh, 1.0 / rh)) < 4.0,
        1.0, 0.0)
    gxi = W - gx
    gyi = H - gy

    def near(u):
        return jnp.where(u % 1.0 < 0.5, 1.0, 0.0) * jnp.where(u > 1.0, 1.0, 0.0)

    jk0, jk1, lm0, lm1 = near(gx), near(gy), near(gxi), near(gyi)
    jmf = jnp.where(o == 0, 1.0,
                    jnp.where(o == 1, jk0,
                              jnp.where(o == 2, jk1,
                                        jnp.where(o == 3, lm0, lm1))))
    validf = jmf * fitf
    valid = validf > 0.5
    ox = jnp.where(o == 1, 1.0, jnp.where(o == 3, -1.0, 0.0))
    oy = jnp.where(o == 2, 1.0, jnp.where(o == 4, -1.0, 0.0))
    gi0 = (gx - ox).astype(jnp.int32)
    gj0 = (gy - oy).astype(jnp.int32)
    gi = jnp.clip(gi0, 0, W - 1)
    gj = jnp.clip(gj0, 0, H - 1)
    bii = bi.astype(jnp.int32)
    clsi = cls_f.astype(jnp.int32)

    if border:
        rowidx = (gj * W + gi) * _NB + bii
    else:
        rowidx = (bii * H + gj) * W + gi
    zero = jnp.zeros((1, _N), jnp.int32)
    aux_ref[...] = jnp.concatenate(
        [rowidx, clsi, zero, zero, zero, zero, zero, zero], axis=0)

    cellf = jnp.where(
        valid,
        (((bii * 3 + a) * H + gj) * W + gi + cell_base).astype(jnp.float32),
        _SENT)
    meta_ref[...] = jnp.concatenate(
        [gx - gi0.astype(jnp.float32), gy - gj0.astype(jnp.float32),
         gw, gh, validf, cellf,
         jnp.broadcast_to(aw, (1, _N)), jnp.broadcast_to(ah, (1, _N))],
        axis=0)


def _prep(tt_pad, s):
    H, W = _HWS[s]
    stride = _IMG // W
    anchors = _ANCH[s * 3:(s + 1) * 3] / stride
    return pl.pallas_call(
        functools.partial(_prep_body, H=H, W=W, anchors=anchors,
                          cell_base=_CELL_BASE[s], border=(s == 2)),
        out_shape=[
            jax.ShapeDtypeStruct((8, _N), jnp.int32),
            jax.ShapeDtypeStruct((8, _N), jnp.float32),
        ],
        in_specs=[pl.BlockSpec((8, _NT), lambda: (0, 0))],
        out_specs=[pl.BlockSpec((8, _N), lambda: (0, 0)),
                   pl.BlockSpec((8, _N), lambda: (0, 0))],
    )(tt_pad)


def _sc_body(t0, t1, t2, x0, x1, x2, m0, m1, m2,
             c0o, c1o, c2o, h0, h1, h2, win,
             rowbuf, clsbuf, databuf, hotbuf, cellbuf, dense, winbuf, gsem):
    wid = lax.axis_index("s") * 2 + lax.axis_index("c")
    tabs = (t0, t1, t2)
    auxs = (x0, x1, x2)
    metas = (m0, m1, m2)
    clsouts = (c0o, c1o, c2o)
    hots = (h0, h1, h2)
    lane = lax.iota(jnp.int32, 16)

    for s_ in range(3):
        for a_ in range(3):
            trips = (40 - wid + 31) // 32

            def chunk(t_, _, s_=s_, a_=a_):
                ci = wid + 32 * t_
                col0 = pl.multiple_of(
                    ((ci // 8) * 3 + a_) * 1024 + (ci % 8) * 128, 128)
                pltpu.sync_copy(auxs[s_].at[0, pl.ds(col0, _CK)], rowbuf)
                pltpu.sync_copy(auxs[s_].at[1, pl.ds(col0, _CK)], clsbuf)
                pltpu.async_copy(tabs[s_].at[rowbuf], databuf, gsem).wait()

                def sub(i, _):
                    q = i * 16 + lane
                    for ch in range(5):
                        v = plsc.load_gather(
                            databuf,
                            [q, jnp.full((16,), a_ * 85 + ch, jnp.int32)])
                        hotbuf[ch, pl.ds(i * 16, 16)] = v
                    cv = clsbuf[pl.ds(i * 16, 16)]
                    v = plsc.load_gather(databuf, [q, a_ * 85 + 5 + cv])
                    hotbuf[5, pl.ds(i * 16, 16)] = v
                    return _

                lax.fori_loop(0, _CK // 16, sub, 0)
                pltpu.sync_copy(databuf, clsouts[s_].at[pl.ds(col0, _CK), :])
                pltpu.sync_copy(hotbuf, hots[s_].at[:, pl.ds(col0, _CK)])
                return _

            lax.fori_loop(0, trips, chunk, 0)

    def ms(i, _):
        dense[pl.ds(i * 16, 16)] = jnp.full((16,), -1, jnp.int32)
        return _

    lax.fori_loop(0, _DCH // 16, ms, 0)

    wbase = wid * _DCH
    for s_ in range(3):
        pltpu.sync_copy(metas[s_].at[5, :], cellbuf)

        def p1b(i, _):
            c = cellbuf[pl.ds(i * 16, 16)].astype(jnp.int32) - wbase
            m = (c >= 0) & (c < _DCH)
            cs = jnp.where(m, c, 0)
            plsc.store_scatter(dense, [cs], i * 16 + lane, mask=m)
            return _

        lax.fori_loop(0, _N // 16, p1b, 0)

    for s_ in range(3):
        pltpu.sync_copy(metas[s_].at[5, :], cellbuf)

        def p2b(i, _):
            c = cellbuf[pl.ds(i * 16, 16)].astype(jnp.int32) - wbase
            m = (c >= 0) & (c < _DCH)
            cs = jnp.where(m, c, 0)
            w = plsc.load_gather(dense, [cs], mask=m)
            isw = m & (w == i * 16 + lane)
            winbuf[pl.ds(i * 16, 16)] = jnp.where(isw, 1.0, 0.0)
            return _

        lax.fori_loop(0, _N // 16, p2b, 0)
        pltpu.sync_copy(winbuf, win.at[wid, pl.ds(s_ * _N, _N)])


def _sc_call(tabs, auxs, metas):
    mesh = plsc.VectorSubcoreMesh(core_axis_name="c", subcore_axis_name="s",
                                  num_cores=2, num_subcores=16)
    f = pl.kernel(
        _sc_body,
        out_type=[
            jax.ShapeDtypeStruct((_N, 256), jnp.float32),
            jax.ShapeDtypeStruct((_N, 256), jnp.float32),
            jax.ShapeDtypeStruct((_N, 256), jnp.float32),
            jax.ShapeDtypeStruct((8, _N), jnp.float32),
            jax.ShapeDtypeStruct((8, _N), jnp.float32),
            jax.ShapeDtypeStruct((8, _N), jnp.float32),
            jax.ShapeDtypeStruct((_NW, 3 * _N), jnp.float32),
        ],
        mesh=mesh,
        scratch_types=[
            pltpu.VMEM((_CK,), jnp.int32),
            pltpu.VMEM((_CK,), jnp.int32),
            pltpu.VMEM((_CK, 256), jnp.float32),
            pltpu.VMEM((8, _CK), jnp.float32),
            pltpu.VMEM((_N,), jnp.float32),
            pltpu.VMEM((_DCH,), jnp.int32),
            pltpu.VMEM((_N,), jnp.float32),
            pltpu.SemaphoreType.DMA,
        ],
        compiler_params=pltpu.CompilerParams(needs_layout_passes=False),
    )
    return f(*tabs, *auxs, *metas)


def _atan_pos(x):
    inv = x > 1.0
    y = jnp.where(inv, 1.0 / x, x)
    z = y * y
    p = y * (0.9998660 + z * (-0.3302995 + z * (0.1801410 + z *
             (-0.0851330 + z * 0.0208351))))
    return jnp.where(inv, (np.pi / 2) - p, p)


def _math_body(c0, c1, c2, h0, h1, h2, m0, m1, m2, w0, w1, w2, out_ref):
    i = pl.program_id(0)

    @pl.when(i == 0)
    def _():
        out_ref[...] = jnp.zeros_like(out_ref)

    a_dyn = i % 3
    acc = jnp.zeros((8, 128), jnp.float32)
    r_i = lax.broadcasted_iota(jnp.int32, (8, 128), 0)
    c_i = lax.broadcasted_iota(jnp.int32, (8, 128), 1)
    for s_, (cb, hot, mt, w) in enumerate(((c0, h0, m0, w0), (c1, h1, m1, w1),
                                           (c2, h2, m2, w2))):
        tbx = mt[0:1, :]
        tby = mt[1:2, :]
        tbw = mt[2:3, :]
        tbh = mt[3:4, :]
        valid = mt[4:5, :]
        aw = mt[6:7, :]
        ah = mt[7:8, :]

        b1x = jax.nn.sigmoid(hot[0:1, :])
        b1y = jax.nn.sigmoid(hot[1:2, :])
        w1_ = jnp.exp(hot[2:3, :]) * aw
        h1_ = jnp.exp(hot[3:4, :]) * ah
        ps4 = hot[4:5, :]
        pstc = hot[5:6, :]

        b1x1 = b1x - w1_ / 2
        b1x2 = b1x + w1_ / 2
        b1y1 = b1y - h1_ / 2
        b1y2 = b1y + h1_ / 2
        b2x1 = tbx - tbw / 2
        b2x2 = tbx + tbw / 2
        b2y1 = tby - tbh / 2
        b2y2 = tby + tbh / 2
        inter = (jnp.maximum(jnp.minimum(b1x2, b2x2) -
                             jnp.maximum(b1x1, b2x1), 0.0) *
                 jnp.maximum(jnp.minimum(b1y2, b2y2) -
                             jnp.maximum(b1y1, b2y1), 0.0))
        union = w1_ * h1_ + tbw * tbh - inter + 1e-16
        iou0 = inter / union
        cw = jnp.maximum(b1x2, b2x2) - jnp.minimum(b1x1, b2x1)
        ch = jnp.maximum(b1y2, b2y2) - jnp.minimum(b1y1, b2y1)
        c2_ = cw * cw + ch * ch + 1e-16
        rho2 = ((b2x1 + b2x2 - b1x1 - b1x2) ** 2 +
                (b2y1 + b2y2 - b1y1 - b1y2) ** 2) / 4
        v = (4.0 / 3.14159 ** 2) * (_atan_pos(tbw / tbh) -
                                    _atan_pos(w1_ / h1_)) ** 2
        alpha = v / (v - iou0 + (1.0 + 1e-16))
        iou = iou0 - (rho2 / c2_ + v * alpha)

        box_p = jnp.sum((1.0 - iou) * valid)
        cnt_p = jnp.sum(valid)

        sp = jnp.logaddexp(0.0, cb[...])
        s0 = jnp.sum(sp[:, 5:85], axis=1, keepdims=True)
        s1 = jnp.sum(sp[:, 90:170], axis=1, keepdims=True)
        s2 = jnp.sum(sp[:, 175:255], axis=1, keepdims=True)
        scol = jnp.where(a_dyn == 0, s0, jnp.where(a_dyn == 1, s1, s2))
        cls_p = jnp.dot(valid, scol)[0, 0] - jnp.sum(pstc * valid)

        wsum = jnp.sum(w[...], axis=0, keepdims=True)
        win_p = jnp.sum(wsum * jnp.maximum(iou, 0.0) * ps4)

        vals = jnp.where(c_i == 0, box_p,
                         jnp.where(c_i == 1, cnt_p,
                                   jnp.where(c_i == 2, cls_p, win_p)))
        acc += jnp.where((r_i == s_) & (c_i < 4), vals, 0.0)

    out_ref[...] += acc


def _math(clss, hots, metas, win):
    nblk = 15
    bw = _N // nblk
    return pl.pallas_call(
        _math_body,
        grid=(nblk,),
        out_shape=jax.ShapeDtypeStruct((8, 128), jnp.float32),
        in_specs=(
            [pl.BlockSpec((bw, 256), lambda i: (i, 0)) for _ in range(3)] +
            [pl.BlockSpec((8, bw), lambda i: (0, i)) for _ in range(3)] +
            [pl.BlockSpec((8, bw), lambda i: (0, i)) for _ in range(3)] +
            [pl.BlockSpec((_NW, bw), lambda i, s_=s_: (0, s_ * nblk + i))
             for s_ in range(3)]),
        out_specs=pl.BlockSpec((8, 128), lambda i: (0, 0)),
    )(*clss, *hots, *metas, win, win, win)


def kernel(pred0, pred1, pred2, targets):
    preds = (pred0, pred1, pred2)
    tt = jnp.pad(targets.T, ((0, 2), (0, 0)))

    auxs, metas = [], []
    for s in range(3):
        aux, meta = _prep(tt, s)
        auxs.append(aux)
        metas.append(meta)

    tabs, objs = [], []
    for s in range(3):
        p_cl = jnp.transpose(preds[s], _PERMS[s])
        tab, obj = _repack(p_cl, s)
        tabs.append(tab)
        objs.append(obj)

    c0o, c1o, c2o, h0, h1, h2, win = _sc_call(tabs, auxs, metas)

    res = _math((c0o, c1o, c2o), (h0, h1, h2), metas, win)

    lbox = jnp.float32(0.0)
    lobj = jnp.float32(0.0)
    lcls = jnp.float32(0.0)
    for s in range(3):
        H, W = _HWS[s]
        box_p, cnt, cls_p, win_p = res[s, 0], res[s, 1], res[s, 2], res[s, 3]
        lbox += box_p / cnt
        lcls += cls_p / (cnt * _NC)
        lobj += (objs[s][0, 0] - win_p) / (_NB * 3 * H * W)
    lbox *= 0.05
    lcls *= 0.5
    loss = lbox + lobj + lcls
    return loss, jnp.stack([lbox, lobj, lcls])


# speedup vs baseline: 3.3390x; 1.4813x over previous
import functools

import numpy as np
import jax
import jax.numpy as jnp
from jax import lax
from jax.experimental import pallas as pl
from jax.experimental.pallas import tpu as pltpu
from jax.experimental.pallas import tpu_sc as plsc

_NC = 80
_IMG = 640
_NB = 32
_NT = 1024
_N = 15360
_ANCH = np.array(
    [[10.0, 13.0], [16.0, 30.0], [33.0, 23.0], [30.0, 61.0], [62.0, 45.0],
     [59.0, 119.0], [116.0, 90.0], [156.0, 198.0], [373.0, 326.0]],
    dtype=np.float32)
_HWS = [(80, 80), (40, 40), (20, 20)]
_CELL_BASE = [0, _NB * 3 * 6400, _NB * 3 * 6400 + _NB * 3 * 1600]
_DTOT = _NB * 3 * (6400 + 1600 + 400)
_SENT = 4.0e6

_NW = 32
_DCH = _DTOT // _NW
_CK = 128
_PERMS = [(0, 2, 3, 1), (0, 2, 3, 1), (2, 3, 0, 1)]


def _repack_body(p_ref, tab_ref, obj_ref, *, bh, W):
    i = pl.program_id(0)

    @pl.when(i == 0)
    def _():
        obj_ref[...] = jnp.zeros_like(obj_ref)

    x = p_ref[0]
    x2 = x.reshape(bh * W, 255)
    tab_ref[...] = jnp.concatenate(
        [x2, jnp.zeros((bh * W, 1), jnp.float32)], axis=1)
    s = (jnp.sum(jnp.logaddexp(0.0, x2[:, 4:5])) +
         jnp.sum(jnp.logaddexp(0.0, x2[:, 89:90])) +
         jnp.sum(jnp.logaddexp(0.0, x2[:, 174:175])))
    r = lax.broadcasted_iota(jnp.int32, (8, 128), 0)
    c = lax.broadcasted_iota(jnp.int32, (8, 128), 1)
    obj_ref[...] += jnp.where((r == 0) & (c == 0), s, 0.0)


def _repack(p_cl, s):
    d0, d1, d2 = p_cl.shape[0], p_cl.shape[1], p_cl.shape[2]
    R = d0 * d1 * d2
    return pl.pallas_call(
        functools.partial(_repack_body, bh=d1, W=d2),
        grid=(d0,),
        out_shape=[jax.ShapeDtypeStruct((R, 256), jnp.float32),
                   jax.ShapeDtypeStruct((8, 128), jnp.float32)],
        in_specs=[pl.BlockSpec((1, d1, d2, 255), lambda i: (i, 0, 0, 0))],
        out_specs=[pl.BlockSpec((d1 * d2, 256), lambda i: (i, 0)),
                   pl.BlockSpec((8, 128), lambda i: (0, 0))],
    )(p_cl)


def _prep_body(tt_ref, aux_ref, meta_ref, *, H, W, anchors, cell_base, border):
    col = lax.broadcasted_iota(jnp.int32, (1, _N), 1)
    a = (col // _NT) % 3
    o = col // (3 * _NT)

    def tiled(r):
        row = tt_ref[r:r + 1, :]
        return jnp.concatenate([row] * 15, axis=1)

    bi = tiled(0)
    cls_f = tiled(1)
    gx = tiled(2) * W
    gy = tiled(3) * H
    gw = tiled(4) * W
    gh = tiled(5) * H

    af = a.astype(jnp.float32)
    aw = jnp.where(af == 0.0, anchors[0, 0],
                   jnp.where(af == 1.0, anchors[1, 0], anchors[2, 0]))
    ah = jnp.where(af == 0.0, anchors[0, 1],
                   jnp.where(af == 1.0, anchors[1, 1], anchors[2, 1]))
    rw = gw / aw
    rh = gh / ah
    fitf = jnp.where(
        jnp.maximum(jnp.maximum(rw, 1.0 / rw), jnp.maximum(rh, 1.0 / rh)) < 4.0,
        1.0, 0.0)
    gxi = W - gx
    gyi = H - gy

    def near(u):
        return jnp.where(u % 1.0 < 0.5, 1.0, 0.0) * jnp.where(u > 1.0, 1.0, 0.0)

    jk0, jk1, lm0, lm1 = near(gx), near(gy), near(gxi), near(gyi)
    jmf = jnp.where(o == 0, 1.0,
                    jnp.where(o == 1, jk0,
                              jnp.where(o == 2, jk1,
                                        jnp.where(o == 3, lm0, lm1))))
    validf = jmf * fitf
    valid = validf > 0.5
    ox = jnp.where(o == 1, 1.0, jnp.where(o == 3, -1.0, 0.0))
    oy = jnp.where(o == 2, 1.0, jnp.where(o == 4, -1.0, 0.0))
    gi0 = (gx - ox).astype(jnp.int32)
    gj0 = (gy - oy).astype(jnp.int32)
    gi = jnp.clip(gi0, 0, W - 1)
    gj = jnp.clip(gj0, 0, H - 1)
    bii = bi.astype(jnp.int32)
    clsi = cls_f.astype(jnp.int32)

    if border:
        rowidx = (gj * W + gi) * _NB + bii
    else:
        rowidx = (bii * H + gj) * W + gi
    zero = jnp.zeros((1, _N), jnp.int32)
    aux_ref[...] = jnp.concatenate(
        [rowidx, clsi, zero, zero, zero, zero, zero, zero], axis=0)

    cellf = jnp.where(
        valid,
        (((bii * 3 + a) * H + gj) * W + gi + cell_base).astype(jnp.float32),
        _SENT)
    meta_ref[...] = jnp.concatenate(
        [gx - gi0.astype(jnp.float32), gy - gj0.astype(jnp.float32),
         gw, gh, validf, cellf,
         jnp.broadcast_to(aw, (1, _N)), jnp.broadcast_to(ah, (1, _N))],
        axis=0)


def _prep(tt_pad, s):
    H, W = _HWS[s]
    stride = _IMG // W
    anchors = _ANCH[s * 3:(s + 1) * 3] / stride
    return pl.pallas_call(
        functools.partial(_prep_body, H=H, W=W, anchors=anchors,
                          cell_base=_CELL_BASE[s], border=(s == 2)),
        out_shape=[
            jax.ShapeDtypeStruct((8, _N), jnp.int32),
            jax.ShapeDtypeStruct((8, _N), jnp.float32),
        ],
        in_specs=[pl.BlockSpec((8, _NT), lambda: (0, 0))],
        out_specs=[pl.BlockSpec((8, _N), lambda: (0, 0)),
                   pl.BlockSpec((8, _N), lambda: (0, 0))],
    )(tt_pad)


def _sc_gather_body(t0, t1, t2, x0, x1, x2,
                    c0o, c1o, c2o, h0, h1, h2,
                    rowbuf, clsbuf, databuf, hotbuf, gsem):
    wid = lax.axis_index("s") * 2 + lax.axis_index("c")
    tabs = (t0, t1, t2)
    auxs = (x0, x1, x2)
    clsouts = (c0o, c1o, c2o)
    hots = (h0, h1, h2)
    lane = lax.iota(jnp.int32, 16)

    for s_ in range(3):
        for a_ in range(3):
            trips = (40 - wid + 31) // 32

            def chunk(t_, _, s_=s_, a_=a_):
                ci = wid + 32 * t_
                col0 = pl.multiple_of(
                    ((ci // 8) * 3 + a_) * 1024 + (ci % 8) * 128, 128)
                pltpu.sync_copy(auxs[s_].at[0, pl.ds(col0, _CK)], rowbuf)
                pltpu.sync_copy(auxs[s_].at[1, pl.ds(col0, _CK)], clsbuf)
                pltpu.async_copy(tabs[s_].at[rowbuf], databuf, gsem).wait()

                def sub(i, _):
                    q = i * 16 + lane
                    for ch in range(5):
                        v = plsc.load_gather(
                            databuf,
                            [q, jnp.full((16,), a_ * 85 + ch, jnp.int32)])
                        hotbuf[ch, pl.ds(i * 16, 16)] = v
                    cv = clsbuf[pl.ds(i * 16, 16)]
                    v = plsc.load_gather(databuf, [q, a_ * 85 + 5 + cv])
                    hotbuf[5, pl.ds(i * 16, 16)] = v
                    return _

                lax.fori_loop(0, _CK // 16, sub, 0)
                pltpu.sync_copy(databuf, clsouts[s_].at[pl.ds(col0, _CK), :])
                pltpu.sync_copy(hotbuf, hots[s_].at[:, pl.ds(col0, _CK)])
                return _

            lax.fori_loop(0, trips, chunk, 0)


def _sc_dedup_body(m0, m1, m2, win, cellbuf, dense, winbuf):
    wid = lax.axis_index("s") * 2 + lax.axis_index("c")
    metas = (m0, m1, m2)
    lane = lax.iota(jnp.int32, 16)

    def ms(i, _):
        dense[pl.ds(i * 16, 16)] = jnp.full((16,), -1, jnp.int32)
        return _

    lax.fori_loop(0, _DCH // 16, ms, 0)

    wbase = wid * _DCH
    for s_ in range(3):
        pltpu.sync_copy(metas[s_].at[5, :], cellbuf)

        def p1b(i, _):
            c = cellbuf[pl.ds(i * 16, 16)].astype(jnp.int32) - wbase
            m = (c >= 0) & (c < _DCH)
            cs = jnp.where(m, c, 0)
            plsc.store_scatter(dense, [cs], i * 16 + lane, mask=m)
            return _

        lax.fori_loop(0, _N // 16, p1b, 0)

    for s_ in range(3):
        pltpu.sync_copy(metas[s_].at[5, :], cellbuf)

        def p2b(i, _):
            c = cellbuf[pl.ds(i * 16, 16)].astype(jnp.int32) - wbase
            m = (c >= 0) & (c < _DCH)
            cs = jnp.where(m, c, 0)
            w = plsc.load_gather(dense, [cs], mask=m)
            isw = m & (w == i * 16 + lane)
            winbuf[pl.ds(i * 16, 16)] = jnp.where(isw, 1.0, 0.0)
            return _

        lax.fori_loop(0, _N // 16, p2b, 0)
        pltpu.sync_copy(winbuf, win.at[wid, pl.ds(s_ * _N, _N)])


def _sc_mesh():
    return plsc.VectorSubcoreMesh(core_axis_name="c", subcore_axis_name="s",
                                  num_cores=2, num_subcores=16)


def _sc_dedup(metas):
    f = pl.kernel(
        _sc_dedup_body,
        out_type=[jax.ShapeDtypeStruct((_NW, 3 * _N), jnp.float32)],
        mesh=_sc_mesh(),
        scratch_types=[
            pltpu.VMEM((_N,), jnp.float32),
            pltpu.VMEM((_DCH,), jnp.int32),
            pltpu.VMEM((_N,), jnp.float32),
        ],
        compiler_params=pltpu.CompilerParams(needs_layout_passes=False),
    )
    return f(*metas)


def _sc_gather(tabs, auxs):
    f = pl.kernel(
        _sc_gather_body,
        out_type=[
            jax.ShapeDtypeStruct((_N, 256), jnp.float32),
            jax.ShapeDtypeStruct((_N, 256), jnp.float32),
            jax.ShapeDtypeStruct((_N, 256), jnp.float32),
            jax.ShapeDtypeStruct((8, _N), jnp.float32),
            jax.ShapeDtypeStruct((8, _N), jnp.float32),
            jax.ShapeDtypeStruct((8, _N), jnp.float32),
        ],
        mesh=_sc_mesh(),
        scratch_types=[
            pltpu.VMEM((_CK,), jnp.int32),
            pltpu.VMEM((_CK,), jnp.int32),
            pltpu.VMEM((_CK, 256), jnp.float32),
            pltpu.VMEM((8, _CK), jnp.float32),
            pltpu.SemaphoreType.DMA,
        ],
        compiler_params=pltpu.CompilerParams(needs_layout_passes=False),
    )
    return f(*tabs, *auxs)


def _atan_pos(x):
    inv = x > 1.0
    y = jnp.where(inv, 1.0 / x, x)
    z = y * y
    p = y * (0.9998660 + z * (-0.3302995 + z * (0.1801410 + z *
             (-0.0851330 + z * 0.0208351))))
    return jnp.where(inv, (np.pi / 2) - p, p)


def _math_body(c0, c1, c2, h0, h1, h2, m0, m1, m2, w0, w1, w2, out_ref):
    i = pl.program_id(0)

    @pl.when(i == 0)
    def _():
        out_ref[...] = jnp.zeros_like(out_ref)

    a_dyn = i % 3
    acc = jnp.zeros((8, 128), jnp.float32)
    r_i = lax.broadcasted_iota(jnp.int32, (8, 128), 0)
    c_i = lax.broadcasted_iota(jnp.int32, (8, 128), 1)
    for s_, (cb, hot, mt, w) in enumerate(((c0, h0, m0, w0), (c1, h1, m1, w1),
                                           (c2, h2, m2, w2))):
        tbx = mt[0:1, :]
        tby = mt[1:2, :]
        tbw = mt[2:3, :]
        tbh = mt[3:4, :]
        valid = mt[4:5, :]
        aw = mt[6:7, :]
        ah = mt[7:8, :]

        b1x = jax.nn.sigmoid(hot[0:1, :])
        b1y = jax.nn.sigmoid(hot[1:2, :])
        w1_ = jnp.exp(hot[2:3, :]) * aw
        h1_ = jnp.exp(hot[3:4, :]) * ah
        ps4 = hot[4:5, :]
        pstc = hot[5:6, :]

        b1x1 = b1x - w1_ / 2
        b1x2 = b1x + w1_ / 2
        b1y1 = b1y - h1_ / 2
        b1y2 = b1y + h1_ / 2
        b2x1 = tbx - tbw / 2
        b2x2 = tbx + tbw / 2
        b2y1 = tby - tbh / 2
        b2y2 = tby + tbh / 2
        inter = (jnp.maximum(jnp.minimum(b1x2, b2x2) -
                             jnp.maximum(b1x1, b2x1), 0.0) *
                 jnp.maximum(jnp.minimum(b1y2, b2y2) -
                             jnp.maximum(b1y1, b2y1), 0.0))
        union = w1_ * h1_ + tbw * tbh - inter + 1e-16
        iou0 = inter / union
        cw = jnp.maximum(b1x2, b2x2) - jnp.minimum(b1x1, b2x1)
        ch = jnp.maximum(b1y2, b2y2) - jnp.minimum(b1y1, b2y1)
        c2_ = cw * cw + ch * ch + 1e-16
        rho2 = ((b2x1 + b2x2 - b1x1 - b1x2) ** 2 +
                (b2y1 + b2y2 - b1y1 - b1y2) ** 2) / 4
        v = (4.0 / 3.14159 ** 2) * (_atan_pos(tbw / tbh) -
                                    _atan_pos(w1_ / h1_)) ** 2
        alpha = v / (v - iou0 + (1.0 + 1e-16))
        iou = iou0 - (rho2 / c2_ + v * alpha)

        box_p = jnp.sum((1.0 - iou) * valid)
        cnt_p = jnp.sum(valid)

        sp = jnp.logaddexp(0.0, cb[...])
        s0 = jnp.sum(sp[:, 5:85], axis=1, keepdims=True)
        s1 = jnp.sum(sp[:, 90:170], axis=1, keepdims=True)
        s2 = jnp.sum(sp[:, 175:255], axis=1, keepdims=True)
        scol = jnp.where(a_dyn == 0, s0, jnp.where(a_dyn == 1, s1, s2))
        cls_p = jnp.dot(valid, scol)[0, 0] - jnp.sum(pstc * valid)

        wsum = jnp.sum(w[...], axis=0, keepdims=True)
        win_p = jnp.sum(wsum * jnp.maximum(iou, 0.0) * ps4)

        vals = jnp.where(c_i == 0, box_p,
                         jnp.where(c_i == 1, cnt_p,
                                   jnp.where(c_i == 2, cls_p, win_p)))
        acc += jnp.where((r_i == s_) & (c_i < 4), vals, 0.0)

    out_ref[...] += acc


def _math(clss, hots, metas, win):
    nblk = 15
    bw = _N // nblk
    return pl.pallas_call(
        _math_body,
        grid=(nblk,),
        out_shape=jax.ShapeDtypeStruct((8, 128), jnp.float32),
        in_specs=(
            [pl.BlockSpec((bw, 256), lambda i: (i, 0)) for _ in range(3)] +
            [pl.BlockSpec((8, bw), lambda i: (0, i)) for _ in range(3)] +
            [pl.BlockSpec((8, bw), lambda i: (0, i)) for _ in range(3)] +
            [pl.BlockSpec((_NW, bw), lambda i, s_=s_: (0, s_ * nblk + i))
             for s_ in range(3)]),
        out_specs=pl.BlockSpec((8, 128), lambda i: (0, 0)),
    )(*clss, *hots, *metas, win, win, win)


def kernel(pred0, pred1, pred2, targets):
    preds = (pred0, pred1, pred2)
    tt = jnp.pad(targets.T, ((0, 2), (0, 0)))

    auxs, metas = [], []
    for s in range(3):
        aux, meta = _prep(tt, s)
        auxs.append(aux)
        metas.append(meta)

    (win,) = _sc_dedup(metas)

    tabs, objs = [], []
    for s in range(3):
        p_cl = jnp.transpose(preds[s], _PERMS[s])
        tab, obj = _repack(p_cl, s)
        tabs.append(tab)
        objs.append(obj)
    c0o, c1o, c2o, h0, h1, h2 = _sc_gather(tabs, auxs)

    res = _math((c0o, c1o, c2o), (h0, h1, h2), metas, win)

    lbox = jnp.float32(0.0)
    lobj = jnp.float32(0.0)
    lcls = jnp.float32(0.0)
    for s in range(3):
        H, W = _HWS[s]
        box_p, cnt, cls_p, win_p = res[s, 0], res[s, 1], res[s, 2], res[s, 3]
        lbox += box_p / cnt
        lcls += cls_p / (cnt * _NC)
        lobj += (objs[s][0, 0] - win_p) / (_NB * 3 * H * W)
    lbox *= 0.05
    lcls *= 0.5
    loss = lbox + lobj + lcls
    return loss, jnp.stack([lbox, lobj, lcls])


# speedup vs baseline: 3.7513x; 1.1235x over previous
import functools

import numpy as np
import jax
import jax.numpy as jnp
from jax import lax
from jax.experimental import pallas as pl
from jax.experimental.pallas import tpu as pltpu
from jax.experimental.pallas import tpu_sc as plsc

_NC = 80
_IMG = 640
_NB = 32
_NT = 1024
_N = 15360
_ANCH = np.array(
    [[10.0, 13.0], [16.0, 30.0], [33.0, 23.0], [30.0, 61.0], [62.0, 45.0],
     [59.0, 119.0], [116.0, 90.0], [156.0, 198.0], [373.0, 326.0]],
    dtype=np.float32)
_HWS = [(80, 80), (40, 40), (20, 20)]
_CELL_BASE = [0, _NB * 3 * 6400, _NB * 3 * 6400 + _NB * 3 * 1600]
_DTOT = _NB * 3 * (6400 + 1600 + 400)
_SENT = 4.0e6

_NW = 32
_DCH = _DTOT // _NW
_CK = 128
_PERMS = [(0, 2, 3, 1), (0, 2, 3, 1), (2, 3, 0, 1)]


def _repack_body(p_ref, tab_ref, obj_ref, *, bh, W):
    i = pl.program_id(0)

    @pl.when(i == 0)
    def _():
        obj_ref[...] = jnp.zeros_like(obj_ref)

    x = p_ref[0]
    x2 = x.reshape(bh * W, 255)
    tab_ref[...] = jnp.concatenate(
        [x2, jnp.zeros((bh * W, 1), jnp.float32)], axis=1)
    s = (jnp.sum(jnp.logaddexp(0.0, x2[:, 4:5])) +
         jnp.sum(jnp.logaddexp(0.0, x2[:, 89:90])) +
         jnp.sum(jnp.logaddexp(0.0, x2[:, 174:175])))
    r = lax.broadcasted_iota(jnp.int32, (8, 128), 0)
    c = lax.broadcasted_iota(jnp.int32, (8, 128), 1)
    obj_ref[...] += jnp.where((r == 0) & (c == 0), s, 0.0)


def _repack(p_cl, s):
    d0, d1, d2 = p_cl.shape[0], p_cl.shape[1], p_cl.shape[2]
    R = d0 * d1 * d2
    return pl.pallas_call(
        functools.partial(_repack_body, bh=d1, W=d2),
        grid=(d0,),
        out_shape=[jax.ShapeDtypeStruct((R, 256), jnp.float32),
                   jax.ShapeDtypeStruct((8, 128), jnp.float32)],
        in_specs=[pl.BlockSpec((1, d1, d2, 255), lambda i: (i, 0, 0, 0))],
        out_specs=[pl.BlockSpec((d1 * d2, 256), lambda i: (i, 0)),
                   pl.BlockSpec((8, 128), lambda i: (0, 0))],
    )(p_cl)


def _prep_body(tt_ref, aux_ref, meta_ref, *, H, W, anchors, cell_base, border):
    col = lax.broadcasted_iota(jnp.int32, (1, _N), 1)
    a = (col // _NT) % 3
    o = col // (3 * _NT)

    def tiled(r):
        row = tt_ref[r:r + 1, :]
        return jnp.concatenate([row] * 15, axis=1)

    bi = tiled(0)
    cls_f = tiled(1)
    gx = tiled(2) * W
    gy = tiled(3) * H
    gw = tiled(4) * W
    gh = tiled(5) * H

    af = a.astype(jnp.float32)
    aw = jnp.where(af == 0.0, anchors[0, 0],
                   jnp.where(af == 1.0, anchors[1, 0], anchors[2, 0]))
    ah = jnp.where(af == 0.0, anchors[0, 1],
                   jnp.where(af == 1.0, anchors[1, 1], anchors[2, 1]))
    rw = gw / aw
    rh = gh / ah
    fitf = jnp.where(
        jnp.maximum(jnp.maximum(rw, 1.0 / rw), jnp.maximum(rh, 1.0 / rh)) < 4.0,
        1.0, 0.0)
    gxi = W - gx
    gyi = H - gy

    def near(u):
        return jnp.where(u % 1.0 < 0.5, 1.0, 0.0) * jnp.where(u > 1.0, 1.0, 0.0)

    jk0, jk1, lm0, lm1 = near(gx), near(gy), near(gxi), near(gyi)
    jmf = jnp.where(o == 0, 1.0,
                    jnp.where(o == 1, jk0,
                              jnp.where(o == 2, jk1,
                                        jnp.where(o == 3, lm0, lm1))))
    validf = jmf * fitf
    valid = validf > 0.5
    ox = jnp.where(o == 1, 1.0, jnp.where(o == 3, -1.0, 0.0))
    oy = jnp.where(o == 2, 1.0, jnp.where(o == 4, -1.0, 0.0))
    gi0 = (gx - ox).astype(jnp.int32)
    gj0 = (gy - oy).astype(jnp.int32)
    gi = jnp.clip(gi0, 0, W - 1)
    gj = jnp.clip(gj0, 0, H - 1)
    bii = bi.astype(jnp.int32)
    clsi = cls_f.astype(jnp.int32)

    if border:
        rowidx = (gj * W + gi) * _NB + bii
    else:
        rowidx = (bii * H + gj) * W + gi
    zero = jnp.zeros((1, _N), jnp.int32)
    aux_ref[...] = jnp.concatenate(
        [rowidx, clsi, zero, zero, zero, zero, zero, zero], axis=0)

    cellf = jnp.where(
        valid,
        (((bii * 3 + a) * H + gj) * W + gi + cell_base).astype(jnp.float32),
        _SENT)
    meta_ref[...] = jnp.concatenate(
        [gx - gi0.astype(jnp.float32), gy - gj0.astype(jnp.float32),
         gw, gh, validf, cellf,
         jnp.broadcast_to(aw, (1, _N)), jnp.broadcast_to(ah, (1, _N))],
        axis=0)


def _prep(tt_pad, s):
    H, W = _HWS[s]
    stride = _IMG // W
    anchors = _ANCH[s * 3:(s + 1) * 3] / stride
    return pl.pallas_call(
        functools.partial(_prep_body, H=H, W=W, anchors=anchors,
                          cell_base=_CELL_BASE[s], border=(s == 2)),
        out_shape=[
            jax.ShapeDtypeStruct((8, _N), jnp.int32),
            jax.ShapeDtypeStruct((8, _N), jnp.float32),
        ],
        in_specs=[pl.BlockSpec((8, _NT), lambda: (0, 0))],
        out_specs=[pl.BlockSpec((8, _N), lambda: (0, 0)),
                   pl.BlockSpec((8, _N), lambda: (0, 0))],
    )(tt_pad)


def _sc_gather_body(tab, aux, clsout, hot,
                    rowbuf, clsbuf, databuf, hotbuf, gsem):
    wid = lax.axis_index("s") * 2 + lax.axis_index("c")
    lane = lax.iota(jnp.int32, 16)
    trips = (120 - wid + 31) // 32

    def chunk(t_, _):
        ci = wid + 32 * t_
        a_ = (ci // 8) % 3
        col0 = pl.multiple_of(ci * _CK, 128)
        pltpu.sync_copy(aux.at[0, pl.ds(col0, _CK)], rowbuf)
        pltpu.sync_copy(aux.at[1, pl.ds(col0, _CK)], clsbuf)
        pltpu.async_copy(tab.at[rowbuf], databuf, gsem).wait()

        def sub(i, _):
            q = i * 16 + lane
            for ch in range(5):
                v = plsc.load_gather(
                    databuf, [q, a_ * 85 + jnp.full((16,), ch, jnp.int32)])
                hotbuf[ch, pl.ds(i * 16, 16)] = v
            cv = clsbuf[pl.ds(i * 16, 16)]
            v = plsc.load_gather(databuf, [q, a_ * 85 + 5 + cv])
            hotbuf[5, pl.ds(i * 16, 16)] = v
            return _

        lax.fori_loop(0, _CK // 16, sub, 0)
        pltpu.sync_copy(databuf, clsout.at[pl.ds(col0, _CK), :])
        pltpu.sync_copy(hotbuf, hot.at[:, pl.ds(col0, _CK)])
        return _

    lax.fori_loop(0, trips, chunk, 0)


def _sc_dedup_body(m0, m1, m2, win, cellbuf, dense, winbuf):
    wid = lax.axis_index("s") * 2 + lax.axis_index("c")
    metas = (m0, m1, m2)
    lane = lax.iota(jnp.int32, 16)

    def ms(i, _):
        dense[pl.ds(i * 16, 16)] = jnp.full((16,), -1, jnp.int32)
        return _

    lax.fori_loop(0, _DCH // 16, ms, 0)

    wbase = wid * _DCH
    for s_ in range(3):
        pltpu.sync_copy(metas[s_].at[5, :], cellbuf)

        def p1b(i, _):
            c = cellbuf[pl.ds(i * 16, 16)].astype(jnp.int32) - wbase
            m = (c >= 0) & (c < _DCH)
            cs = jnp.where(m, c, 0)
            plsc.store_scatter(dense, [cs], i * 16 + lane, mask=m)
            return _

        lax.fori_loop(0, _N // 16, p1b, 0)

    for s_ in range(3):
        pltpu.sync_copy(metas[s_].at[5, :], cellbuf)

        def p2b(i, _):
            c = cellbuf[pl.ds(i * 16, 16)].astype(jnp.int32) - wbase
            m = (c >= 0) & (c < _DCH)
            cs = jnp.where(m, c, 0)
            w = plsc.load_gather(dense, [cs], mask=m)
            isw = m & (w == i * 16 + lane)
            winbuf[pl.ds(i * 16, 16)] = jnp.where(isw, 1.0, 0.0)
            return _

        lax.fori_loop(0, _N // 16, p2b, 0)
        pltpu.sync_copy(winbuf, win.at[wid, pl.ds(s_ * _N, _N)])


def _sc_mesh():
    return plsc.VectorSubcoreMesh(core_axis_name="c", subcore_axis_name="s",
                                  num_cores=2, num_subcores=16)


def _sc_dedup(metas):
    f = pl.kernel(
        _sc_dedup_body,
        out_type=[jax.ShapeDtypeStruct((_NW, 3 * _N), jnp.float32)],
        mesh=_sc_mesh(),
        scratch_types=[
            pltpu.VMEM((_N,), jnp.float32),
            pltpu.VMEM((_DCH,), jnp.int32),
            pltpu.VMEM((_N,), jnp.float32),
        ],
        compiler_params=pltpu.CompilerParams(needs_layout_passes=False),
    )
    return f(*metas)


def _sc_gather(tab, aux):
    f = pl.kernel(
        _sc_gather_body,
        out_type=[
            jax.ShapeDtypeStruct((_N, 256), jnp.float32),
            jax.ShapeDtypeStruct((8, _N), jnp.float32),
        ],
        mesh=_sc_mesh(),
        scratch_types=[
            pltpu.VMEM((_CK,), jnp.int32),
            pltpu.VMEM((_CK,), jnp.int32),
            pltpu.VMEM((_CK, 256), jnp.float32),
            pltpu.VMEM((8, _CK), jnp.float32),
            pltpu.SemaphoreType.DMA,
        ],
        compiler_params=pltpu.CompilerParams(needs_layout_passes=False),
    )
    return f(tab, aux)


def _atan_pos(x):
    inv = x > 1.0
    y = jnp.where(inv, 1.0 / x, x)
    z = y * y
    p = y * (0.9998660 + z * (-0.3302995 + z * (0.1801410 + z *
             (-0.0851330 + z * 0.0208351))))
    return jnp.where(inv, (np.pi / 2) - p, p)


def _math_body(c0, c1, c2, h0, h1, h2, m0, m1, m2, w0, w1, w2, out_ref):
    i = pl.program_id(0)

    @pl.when(i == 0)
    def _():
        out_ref[...] = jnp.zeros_like(out_ref)

    a_dyn = i % 3
    acc = jnp.zeros((8, 128), jnp.float32)
    r_i = lax.broadcasted_iota(jnp.int32, (8, 128), 0)
    c_i = lax.broadcasted_iota(jnp.int32, (8, 128), 1)
    for s_, (cb, hot, mt, w) in enumerate(((c0, h0, m0, w0), (c1, h1, m1, w1),
                                           (c2, h2, m2, w2))):
        tbx = mt[0:1, :]
        tby = mt[1:2, :]
        tbw = mt[2:3, :]
        tbh = mt[3:4, :]
        valid = mt[4:5, :]
        aw = mt[6:7, :]
        ah = mt[7:8, :]

        b1x = jax.nn.sigmoid(hot[0:1, :])
        b1y = jax.nn.sigmoid(hot[1:2, :])
        w1_ = jnp.exp(hot[2:3, :]) * aw
        h1_ = jnp.exp(hot[3:4, :]) * ah
        ps4 = hot[4:5, :]
        pstc = hot[5:6, :]

        b1x1 = b1x - w1_ / 2
        b1x2 = b1x + w1_ / 2
        b1y1 = b1y - h1_ / 2
        b1y2 = b1y + h1_ / 2
        b2x1 = tbx - tbw / 2
        b2x2 = tbx + tbw / 2
        b2y1 = tby - tbh / 2
        b2y2 = tby + tbh / 2
        inter = (jnp.maximum(jnp.minimum(b1x2, b2x2) -
                             jnp.maximum(b1x1, b2x1), 0.0) *
                 jnp.maximum(jnp.minimum(b1y2, b2y2) -
                             jnp.maximum(b1y1, b2y1), 0.0))
        union = w1_ * h1_ + tbw * tbh - inter + 1e-16
        iou0 = inter / union
        cw = jnp.maximum(b1x2, b2x2) - jnp.minimum(b1x1, b2x1)
        ch = jnp.maximum(b1y2, b2y2) - jnp.minimum(b1y1, b2y1)
        c2_ = cw * cw + ch * ch + 1e-16
        rho2 = ((b2x1 + b2x2 - b1x1 - b1x2) ** 2 +
                (b2y1 + b2y2 - b1y1 - b1y2) ** 2) / 4
        v = (4.0 / 3.14159 ** 2) * (_atan_pos(tbw / tbh) -
                                    _atan_pos(w1_ / h1_)) ** 2
        alpha = v / (v - iou0 + (1.0 + 1e-16))
        iou = iou0 - (rho2 / c2_ + v * alpha)

        box_p = jnp.sum((1.0 - iou) * valid)
        cnt_p = jnp.sum(valid)

        sp = jnp.logaddexp(0.0, cb[...])
        s0 = jnp.sum(sp[:, 5:85], axis=1, keepdims=True)
        s1 = jnp.sum(sp[:, 90:170], axis=1, keepdims=True)
        s2 = jnp.sum(sp[:, 175:255], axis=1, keepdims=True)
        scol = jnp.where(a_dyn == 0, s0, jnp.where(a_dyn == 1, s1, s2))
        cls_p = jnp.dot(valid, scol)[0, 0] - jnp.sum(pstc * valid)

        wsum = jnp.sum(w[...], axis=0, keepdims=True)
        win_p = jnp.sum(wsum * jnp.maximum(iou, 0.0) * ps4)

        vals = jnp.where(c_i == 0, box_p,
                         jnp.where(c_i == 1, cnt_p,
                                   jnp.where(c_i == 2, cls_p, win_p)))
        acc += jnp.where((r_i == s_) & (c_i < 4), vals, 0.0)

    out_ref[...] += acc


def _math(clss, hots, metas, win):
    nblk = 15
    bw = _N // nblk
    return pl.pallas_call(
        _math_body,
        grid=(nblk,),
        out_shape=jax.ShapeDtypeStruct((8, 128), jnp.float32),
        in_specs=(
            [pl.BlockSpec((bw, 256), lambda i: (i, 0)) for _ in range(3)] +
            [pl.BlockSpec((8, bw), lambda i: (0, i)) for _ in range(3)] +
            [pl.BlockSpec((8, bw), lambda i: (0, i)) for _ in range(3)] +
            [pl.BlockSpec((_NW, bw), lambda i, s_=s_: (0, s_ * nblk + i))
             for s_ in range(3)]),
        out_specs=pl.BlockSpec((8, 128), lambda i: (0, 0)),
    )(*clss, *hots, *metas, win, win, win)


def kernel(pred0, pred1, pred2, targets):
    preds = (pred0, pred1, pred2)
    tt = jnp.pad(targets.T, ((0, 2), (0, 0)))

    auxs, metas = [], []
    for s in range(3):
        aux, meta = _prep(tt, s)
        auxs.append(aux)
        metas.append(meta)

    (win,) = _sc_dedup(metas)

    clss, hots, objs = [None] * 3, [None] * 3, [None] * 3
    for s in (2, 1, 0):
        p_cl = jnp.transpose(preds[s], _PERMS[s])
        tab, objs[s] = _repack(p_cl, s)
        clss[s], hots[s] = _sc_gather(tab, auxs[s])

    res = _math(tuple(clss), tuple(hots), metas, win)

    lbox = jnp.float32(0.0)
    lobj = jnp.float32(0.0)
    lcls = jnp.float32(0.0)
    for s in range(3):
        H, W = _HWS[s]
        box_p, cnt, cls_p, win_p = res[s, 0], res[s, 1], res[s, 2], res[s, 3]
        lbox += box_p / cnt
        lcls += cls_p / (cnt * _NC)
        lobj += (objs[s][0, 0] - win_p) / (_NB * 3 * H * W)
    lbox *= 0.05
    lcls *= 0.5
    loss = lbox + lobj + lcls
    return loss, jnp.stack([lbox, lobj, lcls])


# speedup vs baseline: 4.6135x; 1.2298x over previous
import functools

import numpy as np
import jax
import jax.numpy as jnp
from jax import lax
from jax.experimental import pallas as pl
from jax.experimental.pallas import tpu as pltpu
from jax.experimental.pallas import tpu_sc as plsc

_NC = 80
_IMG = 640
_NB = 32
_NT = 1024
_N = 15360
_ANCH = np.array(
    [[10.0, 13.0], [16.0, 30.0], [33.0, 23.0], [30.0, 61.0], [62.0, 45.0],
     [59.0, 119.0], [116.0, 90.0], [156.0, 198.0], [373.0, 326.0]],
    dtype=np.float32)
_HWS = [(80, 80), (40, 40), (20, 20)]
_CELL_BASE = [0, _NB * 3 * 6400, _NB * 3 * 6400 + _NB * 3 * 1600]
_DTOT = _NB * 3 * (6400 + 1600 + 400)
_SENT = 4.0e6

_NW = 32
_DCH = _DTOT // _NW
_CK = 128
_PERMS = [(0, 2, 3, 1), (0, 2, 3, 1), (2, 3, 0, 1)]


def _repack_body(*refs, bh, W):
    p_ref, tab_ref, obj_ref = refs[0], refs[-2], refs[-1]
    i = pl.program_id(0)

    @pl.when(i == 0)
    def _():
        obj_ref[...] = jnp.zeros_like(obj_ref)

    x = p_ref[0]
    x2 = x.reshape(bh * W, 255)
    tab_ref[...] = jnp.concatenate(
        [x2, jnp.zeros((bh * W, 1), jnp.float32)], axis=1)
    s = (jnp.sum(jnp.logaddexp(0.0, x2[:, 4:5])) +
         jnp.sum(jnp.logaddexp(0.0, x2[:, 89:90])) +
         jnp.sum(jnp.logaddexp(0.0, x2[:, 174:175])))
    r = lax.broadcasted_iota(jnp.int32, (8, 128), 0)
    c = lax.broadcasted_iota(jnp.int32, (8, 128), 1)
    obj_ref[...] += jnp.where((r == 0) & (c == 0), s, 0.0)


def _repack(p_cl, s, chain=None):
    d0, d1, d2 = p_cl.shape[0], p_cl.shape[1], p_cl.shape[2]
    R = d0 * d1 * d2
    extra = [] if chain is None else [chain]
    return pl.pallas_call(
        functools.partial(_repack_body, bh=d1, W=d2),
        grid=(d0,),
        out_shape=[jax.ShapeDtypeStruct((R, 256), jnp.float32),
                   jax.ShapeDtypeStruct((8, 128), jnp.float32)],
        in_specs=([pl.BlockSpec((1, d1, d2, 255), lambda i: (i, 0, 0, 0))] +
                  [pl.BlockSpec((8, 128), lambda i: (0, 0))
                   for _ in extra]),
        out_specs=[pl.BlockSpec((d1 * d2, 256), lambda i: (i, 0)),
                   pl.BlockSpec((8, 128), lambda i: (0, 0))],
    )(p_cl, *extra)


def _prep_body(tt_ref, aux_ref, meta_ref, *, H, W, anchors, cell_base, border):
    col = lax.broadcasted_iota(jnp.int32, (1, _N), 1)
    a = (col // _NT) % 3
    o = col // (3 * _NT)

    def tiled(r):
        row = tt_ref[r:r + 1, :]
        return jnp.concatenate([row] * 15, axis=1)

    bi = tiled(0)
    cls_f = tiled(1)
    gx = tiled(2) * W
    gy = tiled(3) * H
    gw = tiled(4) * W
    gh = tiled(5) * H

    af = a.astype(jnp.float32)
    aw = jnp.where(af == 0.0, anchors[0, 0],
                   jnp.where(af == 1.0, anchors[1, 0], anchors[2, 0]))
    ah = jnp.where(af == 0.0, anchors[0, 1],
                   jnp.where(af == 1.0, anchors[1, 1], anchors[2, 1]))
    rw = gw / aw
    rh = gh / ah
    fitf = jnp.where(
        jnp.maximum(jnp.maximum(rw, 1.0 / rw), jnp.maximum(rh, 1.0 / rh)) < 4.0,
        1.0, 0.0)
    gxi = W - gx
    gyi = H - gy

    def near(u):
        return jnp.where(u % 1.0 < 0.5, 1.0, 0.0) * jnp.where(u > 1.0, 1.0, 0.0)

    jk0, jk1, lm0, lm1 = near(gx), near(gy), near(gxi), near(gyi)
    jmf = jnp.where(o == 0, 1.0,
                    jnp.where(o == 1, jk0,
                              jnp.where(o == 2, jk1,
                                        jnp.where(o == 3, lm0, lm1))))
    validf = jmf * fitf
    valid = validf > 0.5
    ox = jnp.where(o == 1, 1.0, jnp.where(o == 3, -1.0, 0.0))
    oy = jnp.where(o == 2, 1.0, jnp.where(o == 4, -1.0, 0.0))
    gi0 = (gx - ox).astype(jnp.int32)
    gj0 = (gy - oy).astype(jnp.int32)
    gi = jnp.clip(gi0, 0, W - 1)
    gj = jnp.clip(gj0, 0, H - 1)
    bii = bi.astype(jnp.int32)
    clsi = cls_f.astype(jnp.int32)

    if border:
        rowidx = (gj * W + gi) * _NB + bii
    else:
        rowidx = (bii * H + gj) * W + gi
    zero = jnp.zeros((1, _N), jnp.int32)
    aux_ref[...] = jnp.concatenate(
        [rowidx, clsi, zero, zero, zero, zero, zero, zero], axis=0)

    cellf = jnp.where(
        valid,
        (((bii * 3 + a) * H + gj) * W + gi + cell_base).astype(jnp.float32),
        _SENT)
    meta_ref[...] = jnp.concatenate(
        [gx - gi0.astype(jnp.float32), gy - gj0.astype(jnp.float32),
         gw, gh, validf, cellf,
         jnp.broadcast_to(aw, (1, _N)), jnp.broadcast_to(ah, (1, _N))],
        axis=0)


def _prep(tt_pad, s):
    H, W = _HWS[s]
    stride = _IMG // W
    anchors = _ANCH[s * 3:(s + 1) * 3] / stride
    return pl.pallas_call(
        functools.partial(_prep_body, H=H, W=W, anchors=anchors,
                          cell_base=_CELL_BASE[s], border=(s == 2)),
        out_shape=[
            jax.ShapeDtypeStruct((8, _N), jnp.int32),
            jax.ShapeDtypeStruct((8, _N), jnp.float32),
        ],
        in_specs=[pl.BlockSpec((8, _NT), lambda: (0, 0))],
        out_specs=[pl.BlockSpec((8, _N), lambda: (0, 0)),
                   pl.BlockSpec((8, _N), lambda: (0, 0))],
    )(tt_pad)


def _sc_gather_body(tab, aux, clsout, hot,
                    rowbuf, clsbuf, databuf, hotbuf, gsem):
    wid = lax.axis_index("s") * 2 + lax.axis_index("c")
    lane = lax.iota(jnp.int32, 16)
    trips = (120 - wid + 31) // 32

    def chunk(t_, _):
        ci = wid + 32 * t_
        a_ = (ci // 8) % 3
        col0 = pl.multiple_of(ci * _CK, 128)
        pltpu.sync_copy(aux.at[0, pl.ds(col0, _CK)], rowbuf)
        pltpu.sync_copy(aux.at[1, pl.ds(col0, _CK)], clsbuf)
        pltpu.async_copy(tab.at[rowbuf], databuf, gsem).wait()

        def sub(i, _):
            q = i * 16 + lane
            for ch in range(5):
                v = plsc.load_gather(
                    databuf, [q, a_ * 85 + jnp.full((16,), ch, jnp.int32)])
                hotbuf[ch, pl.ds(i * 16, 16)] = v
            cv = clsbuf[pl.ds(i * 16, 16)]
            v = plsc.load_gather(databuf, [q, a_ * 85 + 5 + cv])
            hotbuf[5, pl.ds(i * 16, 16)] = v
            return _

        lax.fori_loop(0, _CK // 16, sub, 0)
        pltpu.sync_copy(databuf, clsout.at[pl.ds(col0, _CK), :])
        pltpu.sync_copy(hotbuf, hot.at[:, pl.ds(col0, _CK)])
        return _

    lax.fori_loop(0, trips, chunk, 0)


def _sc_dedup_body(m0, m1, m2, win, cellbuf, dense, winbuf):
    wid = lax.axis_index("s") * 2 + lax.axis_index("c")
    metas = (m0, m1, m2)
    lane = lax.iota(jnp.int32, 16)

    def ms(i, _):
        dense[pl.ds(i * 16, 16)] = jnp.full((16,), -1, jnp.int32)
        return _

    lax.fori_loop(0, _DCH // 16, ms, 0, unroll=4)

    wbase = wid * _DCH
    for s_ in range(3):
        pltpu.sync_copy(metas[s_].at[5, :], cellbuf)

        def p1b(i, _):
            c = cellbuf[pl.ds(i * 16, 16)].astype(jnp.int32) - wbase
            m = (c >= 0) & (c < _DCH)
            cs = jnp.where(m, c, 0)
            plsc.store_scatter(dense, [cs], i * 16 + lane, mask=m)
            return _

        lax.fori_loop(0, _N // 16, p1b, 0, unroll=4)

        def p2b(i, _):
            c = cellbuf[pl.ds(i * 16, 16)].astype(jnp.int32) - wbase
            m = (c >= 0) & (c < _DCH)
            cs = jnp.where(m, c, 0)
            w = plsc.load_gather(dense, [cs], mask=m)
            isw = m & (w == i * 16 + lane)
            winbuf[pl.ds(i * 16, 16)] = jnp.where(isw, 1.0, 0.0)
            return _

        lax.fori_loop(0, _N // 16, p2b, 0, unroll=4)
        pltpu.sync_copy(winbuf, win.at[wid, pl.ds(s_ * _N, _N)])


def _sc_mesh():
    return plsc.VectorSubcoreMesh(core_axis_name="c", subcore_axis_name="s",
                                  num_cores=2, num_subcores=16)


def _sc_g2d_body(tab, aux, m0, m1, m2, clsout, hot, win,
                 rowbuf, clsbuf, databuf, hotbuf, cellbuf, dense, winbuf,
                 gsem):
    _sc_gather_body(tab, aux, clsout, hot, rowbuf, clsbuf, databuf, hotbuf,
                    gsem)
    _sc_dedup_body(m0, m1, m2, win, cellbuf, dense, winbuf)


def _sc_g2d(tab, aux, metas):
    f = pl.kernel(
        _sc_g2d_body,
        out_type=[
            jax.ShapeDtypeStruct((_N, 256), jnp.float32),
            jax.ShapeDtypeStruct((8, _N), jnp.float32),
            jax.ShapeDtypeStruct((_NW, 3 * _N), jnp.float32),
        ],
        mesh=_sc_mesh(),
        scratch_types=[
            pltpu.VMEM((_CK,), jnp.int32),
            pltpu.VMEM((_CK,), jnp.int32),
            pltpu.VMEM((_CK, 256), jnp.float32),
            pltpu.VMEM((8, _CK), jnp.float32),
            pltpu.VMEM((_N,), jnp.float32),
            pltpu.VMEM((_DCH,), jnp.int32),
            pltpu.VMEM((_N,), jnp.float32),
            pltpu.SemaphoreType.DMA,
        ],
        compiler_params=pltpu.CompilerParams(needs_layout_passes=False),
    )
    return f(tab, aux, *metas)


def _sc_gather(tab, aux):
    f = pl.kernel(
        _sc_gather_body,
        out_type=[
            jax.ShapeDtypeStruct((_N, 256), jnp.float32),
            jax.ShapeDtypeStruct((8, _N), jnp.float32),
        ],
        mesh=_sc_mesh(),
        scratch_types=[
            pltpu.VMEM((_CK,), jnp.int32),
            pltpu.VMEM((_CK,), jnp.int32),
            pltpu.VMEM((_CK, 256), jnp.float32),
            pltpu.VMEM((8, _CK), jnp.float32),
            pltpu.SemaphoreType.DMA,
        ],
        compiler_params=pltpu.CompilerParams(needs_layout_passes=False),
    )
    return f(tab, aux)


def _atan_pos(x):
    inv = x > 1.0
    y = jnp.where(inv, 1.0 / x, x)
    z = y * y
    p = y * (0.9998660 + z * (-0.3302995 + z * (0.1801410 + z *
             (-0.0851330 + z * 0.0208351))))
    return jnp.where(inv, (np.pi / 2) - p, p)


def _math_body(c0, c1, c2, h0, h1, h2, m0, m1, m2, w0, w1, w2, out_ref):
    i = pl.program_id(0)

    @pl.when(i == 0)
    def _():
        out_ref[...] = jnp.zeros_like(out_ref)

    a_dyn = i % 3
    acc = jnp.zeros((8, 128), jnp.float32)
    r_i = lax.broadcasted_iota(jnp.int32, (8, 128), 0)
    c_i = lax.broadcasted_iota(jnp.int32, (8, 128), 1)
    for s_, (cb, hot, mt, w) in enumerate(((c0, h0, m0, w0), (c1, h1, m1, w1),
                                           (c2, h2, m2, w2))):
        tbx = mt[0:1, :]
        tby = mt[1:2, :]
        tbw = mt[2:3, :]
        tbh = mt[3:4, :]
        valid = mt[4:5, :]
        aw = mt[6:7, :]
        ah = mt[7:8, :]

        b1x = jax.nn.sigmoid(hot[0:1, :])
        b1y = jax.nn.sigmoid(hot[1:2, :])
        w1_ = jnp.exp(hot[2:3, :]) * aw
        h1_ = jnp.exp(hot[3:4, :]) * ah
        ps4 = hot[4:5, :]
        pstc = hot[5:6, :]

        b1x1 = b1x - w1_ / 2
        b1x2 = b1x + w1_ / 2
        b1y1 = b1y - h1_ / 2
        b1y2 = b1y + h1_ / 2
        b2x1 = tbx - tbw / 2
        b2x2 = tbx + tbw / 2
        b2y1 = tby - tbh / 2
        b2y2 = tby + tbh / 2
        inter = (jnp.maximum(jnp.minimum(b1x2, b2x2) -
                             jnp.maximum(b1x1, b2x1), 0.0) *
                 jnp.maximum(jnp.minimum(b1y2, b2y2) -
                             jnp.maximum(b1y1, b2y1), 0.0))
        union = w1_ * h1_ + tbw * tbh - inter + 1e-16
        iou0 = inter / union
        cw = jnp.maximum(b1x2, b2x2) - jnp.minimum(b1x1, b2x1)
        ch = jnp.maximum(b1y2, b2y2) - jnp.minimum(b1y1, b2y1)
        c2_ = cw * cw + ch * ch + 1e-16
        rho2 = ((b2x1 + b2x2 - b1x1 - b1x2) ** 2 +
                (b2y1 + b2y2 - b1y1 - b1y2) ** 2) / 4
        v = (4.0 / 3.14159 ** 2) * (_atan_pos(tbw / tbh) -
                                    _atan_pos(w1_ / h1_)) ** 2
        alpha = v / (v - iou0 + (1.0 + 1e-16))
        iou = iou0 - (rho2 / c2_ + v * alpha)

        box_p = jnp.sum((1.0 - iou) * valid)
        cnt_p = jnp.sum(valid)

        sp = jnp.logaddexp(0.0, cb[...])
        s0 = jnp.sum(sp[:, 5:85], axis=1, keepdims=True)
        s1 = jnp.sum(sp[:, 90:170], axis=1, keepdims=True)
        s2 = jnp.sum(sp[:, 175:255], axis=1, keepdims=True)
        scol = jnp.where(a_dyn == 0, s0, jnp.where(a_dyn == 1, s1, s2))
        cls_p = jnp.dot(valid, scol)[0, 0] - jnp.sum(pstc * valid)

        wsum = jnp.sum(w[...], axis=0, keepdims=True)
        win_p = jnp.sum(wsum * jnp.maximum(iou, 0.0) * ps4)

        vals = jnp.where(c_i == 0, box_p,
                         jnp.where(c_i == 1, cnt_p,
                                   jnp.where(c_i == 2, cls_p, win_p)))
        acc += jnp.where((r_i == s_) & (c_i < 4), vals, 0.0)

    out_ref[...] += acc


def _math(clss, hots, metas, win):
    nblk = 15
    bw = _N // nblk
    return pl.pallas_call(
        _math_body,
        grid=(nblk,),
        out_shape=jax.ShapeDtypeStruct((8, 128), jnp.float32),
        in_specs=(
            [pl.BlockSpec((bw, 256), lambda i: (i, 0)) for _ in range(3)] +
            [pl.BlockSpec((8, bw), lambda i: (0, i)) for _ in range(3)] +
            [pl.BlockSpec((8, bw), lambda i: (0, i)) for _ in range(3)] +
            [pl.BlockSpec((_NW, bw), lambda i, s_=s_: (0, s_ * nblk + i))
             for s_ in range(3)]),
        out_specs=pl.BlockSpec((8, 128), lambda i: (0, 0)),
    )(*clss, *hots, *metas, win, win, win)


def kernel(pred0, pred1, pred2, targets):
    preds = (pred0, pred1, pred2)
    tt = jnp.pad(targets.T, ((0, 2), (0, 0)))

    auxs, metas = [], []
    for s in range(3):
        aux, meta = _prep(tt, s)
        auxs.append(aux)
        metas.append(meta)

    clss, hots, objs = [None] * 3, [None] * 3, [None] * 3
    views = [jnp.transpose(preds[s], _PERMS[s]) for s in range(3)]
    tab2, objs[2] = _repack(views[2], 2)
    clss[2], hots[2], win = _sc_g2d(tab2, auxs[2], metas)
    tab1, objs[1] = _repack(views[1], 1, chain=objs[2])
    clss[1], hots[1] = _sc_gather(tab1, auxs[1])
    tab0, objs[0] = _repack(views[0], 0, chain=objs[1])
    clss[0], hots[0] = _sc_gather(tab0, auxs[0])

    res = _math(tuple(clss), tuple(hots), metas, win)

    lbox = jnp.float32(0.0)
    lobj = jnp.float32(0.0)
    lcls = jnp.float32(0.0)
    for s in range(3):
        H, W = _HWS[s]
        box_p, cnt, cls_p, win_p = res[s, 0], res[s, 1], res[s, 2], res[s, 3]
        lbox += box_p / cnt
        lcls += cls_p / (cnt * _NC)
        lobj += (objs[s][0, 0] - win_p) / (_NB * 3 * H * W)
    lbox *= 0.05
    lcls *= 0.5
    loss = lbox + lobj + lcls
    return loss, jnp.stack([lbox, lobj, lcls])
